# baseline (speedup 1.0000x reference)
import functools

import jax
import jax.numpy as jnp
from jax import lax
from jax.experimental import pallas as pl
from jax.experimental.pallas import tpu as pltpu
from jax.experimental.pallas import tpu_sc as plsc

_NC = 2
_NS = 16
_NW = _NC * _NS
_C = 80
_NSEG = 4


def _sc_gather(table, idx3):
    nchunks_w = idx3.shape[1]
    n = _NW * nchunks_w * _C
    d = table.shape[1]
    per_w = nchunks_w * _C
    npair = nchunks_w // 2
    mesh = plsc.VectorSubcoreMesh(core_axis_name="c", subcore_axis_name="s")

    @functools.partial(
        pl.kernel,
        out_type=jax.ShapeDtypeStruct((n, d), jnp.float32),
        mesh=mesh,
        scratch_types=[
            pltpu.VMEM((nchunks_w, _C), jnp.int32),
            pltpu.VMEM((_C, d), jnp.float32),
            pltpu.VMEM((_C, d), jnp.float32),
            pltpu.SemaphoreType.DMA,
            pltpu.SemaphoreType.DMA,
        ],
    )
    def gather_kernel(table_hbm, idx_hbm, out_hbm, idxb, r0, r1, s0, s1):
        wid = lax.axis_index("s") * _NC + lax.axis_index("c")
        base = wid * per_w
        pltpu.sync_copy(idx_hbm.at[wid], idxb)

        def start(i, buf, sem):
            pltpu.async_copy(table_hbm.at[idxb.at[i]], buf, sem)

        def wait(buf, sem):
            pltpu.make_async_copy(table_hbm.at[idxb.at[0]], buf, sem).wait()

        def store(i, buf):
            pltpu.sync_copy(buf, out_hbm.at[pl.ds(base + i * _C, _C)])

        start(0, r0, s0)

        def body(j, carry):
            i0 = 2 * j
            i1 = i0 + 1
            start(i1, r1, s1)
            wait(r0, s0)
            store(i0, r0)

            @pl.when(j < npair - 1)
            def _prefetch():
                start(i0 + 2, r0, s0)

            wait(r1, s1)
            store(i1, r1)
            return carry

        lax.fori_loop(0, npair, body, 0)

    return gather_kernel(table, idx3)


def _project_seg(x3, pos, tok, w, b2, out_prev, seg_batch0, bb, full_batch):
    seg_batch, seq, d = x3.shape
    h = w.shape[0]
    off = seg_batch0 // bb

    def body(*refs):
        x_ref, pos_ref, tok_ref, w_ref, b_ref = refs[:5]
        o_ref = refs[-1]
        ptok = pos_ref[...] + tok_ref[...]
        s = (x_ref[...] + ptok[None]).reshape(bb * seq, d).astype(jnp.bfloat16)
        wb = w_ref[...].astype(jnp.bfloat16)
        y = lax.dot_general(
            s, wb,
            dimension_numbers=(((1,), (1,)), ((), ())),
            preferred_element_type=jnp.float32,
        )
        o_ref[...] = (y + b_ref[...]).reshape(bb, seq, h)

    in_specs = [
        pl.BlockSpec((bb, seq, d), lambda i: (i, 0, 0)),
        pl.BlockSpec((seq, d), lambda i: (0, 0)),
        pl.BlockSpec((1, d), lambda i: (0, 0)),
        pl.BlockSpec((h, d), lambda i: (0, 0)),
        pl.BlockSpec((1, h), lambda i: (0, 0)),
    ]
    args = [x3, pos, tok, w, b2]
    aliases = {}
    if out_prev is not None:
        in_specs.append(pl.BlockSpec(memory_space=pl.ANY))
        args.append(out_prev)
        aliases = {5: 0}

    return pl.pallas_call(
        body,
        grid=(seg_batch // bb,),
        in_specs=in_specs,
        out_specs=pl.BlockSpec((bb, seq, h), lambda i: (off + i, 0, 0)),
        out_shape=jax.ShapeDtypeStruct((full_batch, seq, h), jnp.float32),
        input_output_aliases=aliases,
    )(*args)


def kernel(input_ids, word_embeddings, token_type_embeddings, position_embeddings, W, b):
    batch, seq = input_ids.shape
    d = word_embeddings.shape[1]
    h = W.shape[0]
    n = batch * seq
    n_seg = n // _NSEG
    seg_batch = batch // _NSEG
    nchunks_w = n_seg // (_NW * _C)

    ids = input_ids.reshape(-1).astype(jnp.int32)
    pos = position_embeddings[:seq]
    tok = token_type_embeddings[0:1]
    b2 = b.reshape(1, h)

    gathered = [
        _sc_gather(word_embeddings,
                   lax.dynamic_slice_in_dim(ids, k * n_seg, n_seg).reshape(_NW, nchunks_w, _C))
        for k in range(_NSEG)
    ]

    out = None
    for k in range(_NSEG):
        x3 = gathered[k].reshape(seg_batch, seq, d)
        out = _project_seg(x3, pos, tok, W, b2, out, k * seg_batch, 16, batch)
    return out


# speedup vs baseline: 1.0541x; 1.0541x over previous
import functools

import jax
import jax.numpy as jnp
from jax import lax
from jax.experimental import pallas as pl
from jax.experimental.pallas import tpu as pltpu
from jax.experimental.pallas import tpu_sc as plsc

_NC = 2
_NS = 16
_NW = _NC * _NS
_C = 128


def _sc_gather(table, idx3):
    nchunks_w = idx3.shape[1]
    n = _NW * nchunks_w * _C
    d = table.shape[1]
    per_w = nchunks_w * _C
    npair = nchunks_w // 2
    mesh = plsc.VectorSubcoreMesh(core_axis_name="c", subcore_axis_name="s")

    @functools.partial(
        pl.kernel,
        out_type=jax.ShapeDtypeStruct((n, d), jnp.float32),
        mesh=mesh,
        scratch_types=[
            pltpu.VMEM((nchunks_w, _C), jnp.int32),
            pltpu.VMEM((_C, d), jnp.float32),
            pltpu.VMEM((_C, d), jnp.float32),
            pltpu.SemaphoreType.DMA,
            pltpu.SemaphoreType.DMA,
        ],
    )
    def gather_kernel(table_hbm, idx_hbm, out_hbm, idxb, r0, r1, s0, s1):
        wid = lax.axis_index("s") * _NC + lax.axis_index("c")
        base = wid * per_w
        pltpu.sync_copy(idx_hbm.at[wid], idxb)

        def start(i, buf, sem):
            pltpu.async_copy(table_hbm.at[idxb.at[i]], buf, sem)

        def wait(buf, sem):
            pltpu.make_async_copy(table_hbm.at[idxb.at[0]], buf, sem).wait()

        def store(i, buf):
            pltpu.sync_copy(buf, out_hbm.at[pl.ds(base + i * _C, _C)])

        start(0, r0, s0)

        def body(j, carry):
            i0 = 2 * j
            i1 = i0 + 1
            start(i1, r1, s1)
            wait(r0, s0)
            store(i0, r0)

            @pl.when(j < npair - 1)
            def _prefetch():
                start(i0 + 2, r0, s0)

            wait(r1, s1)
            store(i1, r1)
            return carry

        lax.fori_loop(0, npair, body, 0)

    return gather_kernel(table, idx3)


def _project(x3, pos, tok, w, b2, bb):
    batch, seq, d = x3.shape
    h = w.shape[0]

    def body(x_ref, pos_ref, tok_ref, w_ref, b_ref, o_ref):
        ptok = pos_ref[...] + tok_ref[...]
        s = (x_ref[...] + ptok[None]).reshape(bb * seq, d).astype(jnp.bfloat16)
        wb = w_ref[...].astype(jnp.bfloat16)
        y = lax.dot_general(
            s, wb,
            dimension_numbers=(((1,), (1,)), ((), ())),
            preferred_element_type=jnp.float32,
        )
        o_ref[...] = (y + b_ref[...]).reshape(bb, seq, h)

    return pl.pallas_call(
        body,
        grid=(batch // bb,),
        in_specs=[
            pl.BlockSpec((bb, seq, d), lambda i: (i, 0, 0)),
            pl.BlockSpec((seq, d), lambda i: (0, 0)),
            pl.BlockSpec((1, d), lambda i: (0, 0)),
            pl.BlockSpec((h, d), lambda i: (0, 0)),
            pl.BlockSpec((1, h), lambda i: (0, 0)),
        ],
        out_specs=pl.BlockSpec((bb, seq, h), lambda i: (i, 0, 0)),
        out_shape=jax.ShapeDtypeStruct((batch, seq, h), jnp.float32),
    )(x3, pos, tok, w, b2)


def kernel(input_ids, word_embeddings, token_type_embeddings, position_embeddings, W, b):
    batch, seq = input_ids.shape
    d = word_embeddings.shape[1]
    h = W.shape[0]
    n = batch * seq
    idx3 = input_ids.reshape(_NW, n // (_NW * _C), _C).astype(jnp.int32)
    gathered = _sc_gather(word_embeddings, idx3)
    x3 = gathered.reshape(batch, seq, d)
    pos = position_embeddings[:seq]
    tok = token_type_embeddings[0:1]
    b2 = b.reshape(1, h)
    return _project(x3, pos, tok, W, b2, 32)


# speedup vs baseline: 1.0600x; 1.0055x over previous
import functools

import jax
import jax.numpy as jnp
from jax import lax
from jax.experimental import pallas as pl
from jax.experimental.pallas import tpu as pltpu
from jax.experimental.pallas import tpu_sc as plsc

_NC = 2
_NS = 16
_NW = _NC * _NS
_C = 128


def _sc_gather(table, idx3):
    nchunks_w = idx3.shape[1]
    n = _NW * nchunks_w * _C
    d = table.shape[1]
    per_w = nchunks_w * _C
    assert nchunks_w % 4 == 2 and nchunks_w >= 6
    mesh = plsc.VectorSubcoreMesh(core_axis_name="c", subcore_axis_name="s")

    @functools.partial(
        pl.kernel,
        out_type=jax.ShapeDtypeStruct((n, d), jnp.float32),
        mesh=mesh,
        scratch_types=[
            pltpu.VMEM((nchunks_w, _C), jnp.int32),
            pltpu.VMEM((_C, d), jnp.float32),
            pltpu.VMEM((_C, d), jnp.float32),
            pltpu.VMEM((_C, d), jnp.float32),
            pltpu.VMEM((_C, d), jnp.float32),
            pltpu.SemaphoreType.DMA,
            pltpu.SemaphoreType.DMA,
            pltpu.SemaphoreType.DMA,
            pltpu.SemaphoreType.DMA,
            pltpu.SemaphoreType.DMA,
            pltpu.SemaphoreType.DMA,
            pltpu.SemaphoreType.DMA,
            pltpu.SemaphoreType.DMA,
        ],
    )
    def gather_kernel(table_hbm, idx_hbm, out_hbm, idxb,
                      r0, r1, r2, r3, g0, g1, g2, g3, t0, t1, t2, t3):
        wid = lax.axis_index("s") * _NC + lax.axis_index("c")
        base = wid * per_w
        pltpu.sync_copy(idx_hbm.at[wid], idxb)

        bufs = (r0, r1, r2, r3)
        gsem = (g0, g1, g2, g3)
        tsem = (t0, t1, t2, t3)

        def start_g(i, b):
            pltpu.async_copy(table_hbm.at[idxb.at[i]], bufs[b], gsem[b])

        def wait_g(b):
            pltpu.make_async_copy(
                table_hbm.at[idxb.at[0]], bufs[b], gsem[b]).wait()

        def start_st(i, b):
            pltpu.async_copy(bufs[b], out_hbm.at[pl.ds(base + i * _C, _C)],
                             tsem[b])

        def wait_st(b):
            pltpu.make_async_copy(
                bufs[b], out_hbm.at[pl.ds(base, _C)], tsem[b]).wait()

        start_g(0, 0)
        start_g(1, 1)
        wait_g(0); start_st(0, 0); start_g(2, 2)
        wait_g(1); start_st(1, 1); start_g(3, 3)
        wait_g(2); start_st(2, 2); wait_st(0); start_g(4, 0)
        wait_g(3); start_st(3, 3); wait_st(1); start_g(5, 1)

        def body(j, carry):
            for b in range(4):
                i = 4 * j + b
                wait_g(b)
                start_st(i, b)
                bp = (b + 2) % 4
                wait_st(bp)
                start_g(i + 2, bp)
            return carry

        lax.fori_loop(1, (nchunks_w - 2) // 4, body, 0)
        wait_g(0); start_st(nchunks_w - 2, 0)
        wait_g(1); start_st(nchunks_w - 1, 1)
        wait_st(2); wait_st(3); wait_st(0); wait_st(1)

    return gather_kernel(table, idx3)


def _project(x3, pos, tok, w, b2, bb):
    batch, seq, d = x3.shape
    h = w.shape[0]

    def body(x_ref, pos_ref, tok_ref, w_ref, b_ref, o_ref):
        ptok = pos_ref[...] + tok_ref[...]
        s = (x_ref[...] + ptok[None]).reshape(bb * seq, d).astype(jnp.bfloat16)
        wb = w_ref[...].astype(jnp.bfloat16)
        y = lax.dot_general(
            s, wb,
            dimension_numbers=(((1,), (1,)), ((), ())),
            preferred_element_type=jnp.float32,
        )
        o_ref[...] = (y + b_ref[...]).reshape(bb, seq, h)

    return pl.pallas_call(
        body,
        grid=(batch // bb,),
        in_specs=[
            pl.BlockSpec((bb, seq, d), lambda i: (i, 0, 0)),
            pl.BlockSpec((seq, d), lambda i: (0, 0)),
            pl.BlockSpec((1, d), lambda i: (0, 0)),
            pl.BlockSpec((h, d), lambda i: (0, 0)),
            pl.BlockSpec((1, h), lambda i: (0, 0)),
        ],
        out_specs=pl.BlockSpec((bb, seq, h), lambda i: (i, 0, 0)),
        out_shape=jax.ShapeDtypeStruct((batch, seq, h), jnp.float32),
    )(x3, pos, tok, w, b2)


def kernel(input_ids, word_embeddings, token_type_embeddings, position_embeddings, W, b):
    batch, seq = input_ids.shape
    d = word_embeddings.shape[1]
    h = W.shape[0]
    n = batch * seq
    idx3 = input_ids.reshape(_NW, n // (_NW * _C), _C).astype(jnp.int32)
    gathered = _sc_gather(word_embeddings, idx3)
    x3 = gathered.reshape(batch, seq, d)
    pos = position_embeddings[:seq]
    tok = token_type_embeddings[0:1]
    b2 = b.reshape(1, h)
    return _project(x3, pos, tok, W, b2, 32)
